# baseline (speedup 1.0000x reference)
import functools

import jax
import jax.numpy as jnp
from jax import lax
from jax.experimental import pallas as pl
from jax.experimental.pallas import tpu as pltpu
from jax.experimental.pallas import tpu_sc as plsc

N_NODES = 10000
D = 128
N_EDGES = 320000
E_SC = N_EDGES // 2
C = 16000
NCHUNK = E_SC // C
G = C // 16
EPT = C // 16
RG = (EPT + 15) // 16
OB = 16 * RG


def _sc_kernel(zt_hbm, row_hbm, col_hbm, out_hbm,
               zt0, zt1, zt2, zt3, ir, ic,
               part, rb, ob0, ob1, part_sp,
               semi, semz, so0, so1):
    c = lax.axis_index("c")
    s = lax.axis_index("s")
    ebase = c * E_SC
    zts = (zt0, zt1, zt2, zt3)
    obs = (ob0, ob1)
    sos = (so0, so1)

    def issue_idx(ci):
        off = ebase + ci * C
        pltpu.async_copy(row_hbm.at[pl.ds(off, C)], ir, semi)
        pltpu.async_copy(col_hbm.at[pl.ds(off, C)], ic, semi)

    def wait_idx():
        pltpu.make_async_copy(row_hbm.at[pl.ds(0, C)], ir, semi).wait()
        pltpu.make_async_copy(row_hbm.at[pl.ds(0, C)], ic, semi).wait()

    def wait_out(b):
        pltpu.make_async_copy(
            obs[b].at[pl.ds(0, EPT)],
            out_hbm.at[pl.ds(ebase, EPT)], sos[b]).wait()

    def compute():
        @plsc.parallel_loop(0, G, unroll=2)
        def grp(g):
            er = ir[pl.ds(g * 16, 16)]
            ec = ic[pl.ds(g * 16, 16)]
            accs = [None, None]
            for w in range(4):
                ga = plsc.load_gather(zts[w], [er])
                gc = plsc.load_gather(zts[w], [ec])
                p = plsc.bitcast(ga, jnp.bfloat16) * plsc.bitcast(gc, jnp.bfloat16)
                p0, p1 = plsc.unpack(p, format=plsc.PackFormat.INTERLEAVED)
                q = p0 + p1
                accs[w % 2] = q if accs[w % 2] is None else accs[w % 2] + q
            part[pl.ds(g * 16, 16)] = accs[0] + accs[1]

    def reduce_pass(b):
        o = obs[b]

        @plsc.parallel_loop(0, RG)
        def red(k):
            vs = [rb[t, pl.ds(k * 16, 16)] for t in range(16)]
            while len(vs) > 1:
                vs = [vs[i] + vs[i + 1] for i in range(0, len(vs), 2)]
            o[pl.ds(k * 16, 16)] = 1.0 / (1.0 + jnp.exp(-vs[0]))

    def body(ci, b):
        wait_idx()
        compute()

        @pl.when(ci + 1 < NCHUNK)
        def _():
            issue_idx(ci + 1)

        pltpu.sync_copy(part, part_sp.at[0, s])
        plsc.subcore_barrier()
        pltpu.sync_copy(part_sp.at[0, :, pl.ds(s * EPT, EPT)],
                        rb.at[pl.ds(0, 16), :])
        plsc.subcore_barrier()

        @pl.when(ci >= 2)
        def _():
            wait_out(b)

        reduce_pass(b)
        pltpu.async_copy(
            obs[b].at[pl.ds(0, EPT)],
            out_hbm.at[pl.ds(ebase + ci * C + s * EPT, EPT)], sos[b])

    for w in range(4):
        pltpu.async_copy(zt_hbm.at[s, w], zts[w], semz)
    issue_idx(0)
    for w in range(4):
        pltpu.make_async_copy(zt_hbm.at[0, 0], zts[w], semz).wait()

    def pair(si, _):
        for b in (0, 1):
            body(si * 2 + b, b)
        return 0

    lax.fori_loop(0, NCHUNK // 2, pair, 0)
    wait_out(0)
    wait_out(1)


@jax.jit
def kernel(z, edge_index):
    row = edge_index[0].astype(jnp.int32)
    col = edge_index[1].astype(jnp.int32)
    zb = z.astype(jnp.bfloat16).reshape(N_NODES, 16, 4, 2)
    zt = lax.bitcast_convert_type(zb.transpose(1, 2, 0, 3), jnp.int32)
    mesh = plsc.VectorSubcoreMesh(core_axis_name="c", subcore_axis_name="s")
    f = functools.partial(
        pl.kernel,
        mesh=mesh,
        compiler_params=pltpu.CompilerParams(
            needs_layout_passes=False, use_tc_tiling_on_sc=False),
        out_type=jax.ShapeDtypeStruct((N_EDGES,), jnp.float32),
        scratch_types=[
            pltpu.VMEM((N_NODES,), jnp.int32),
            pltpu.VMEM((N_NODES,), jnp.int32),
            pltpu.VMEM((N_NODES,), jnp.int32),
            pltpu.VMEM((N_NODES,), jnp.int32),
            pltpu.VMEM((C,), jnp.int32),
            pltpu.VMEM((C,), jnp.int32),
            pltpu.VMEM((C,), jnp.float32),
            pltpu.VMEM((17, EPT), jnp.float32),
            pltpu.VMEM((OB,), jnp.float32),
            pltpu.VMEM((OB,), jnp.float32),
            pltpu.VMEM_SHARED((1, 16, C), jnp.float32),
            pltpu.SemaphoreType.DMA,
            pltpu.SemaphoreType.DMA,
            pltpu.SemaphoreType.DMA,
            pltpu.SemaphoreType.DMA,
        ],
    )(_sc_kernel)
    return f(zt, row, col)


# speedup vs baseline: 1.3285x; 1.3285x over previous
import functools

import jax
import jax.numpy as jnp
from jax import lax
from jax.experimental import pallas as pl
from jax.experimental.pallas import tpu as pltpu
from jax.experimental.pallas import tpu_sc as plsc

N_NODES = 10000
D = 128
DW = D // 2
N_EDGES = 320000
NW = 32
E_W = N_EDGES // NW
C = 200
NCHUNK = E_W // C
NBLK = C // 16 + 1


def _sc_kernel(z_hbm, row_hbm, col_hbm, out_hbm,
               idxv, zs, ab0, ab1, o0, o1, semi, sg0, sg1, sh0, sh1,
               so0, so1):
    wid = lax.axis_index("s") * 2 + lax.axis_index("c")
    sid = lax.axis_index("s")
    base = wid * E_W
    lanes = lax.iota(jnp.int32, 16)
    last_lane = lanes == 15
    abs_ = (ab0, ab1)
    os_ = (o0, o1)
    sgs = (sg0, sg1)
    sgh = (sh0, sh1)
    sos = (so0, so1)

    def gathers(ci, b):
        pltpu.async_copy(
            zs.at[idxv.at[pl.ds(ci * C, C)]],
            abs_[b].at[pl.ds(0, C)], sgh[b])
        pltpu.async_copy(
            zs.at[idxv.at[pl.ds(E_W + ci * C, C)]],
            abs_[b].at[pl.ds(C, C)], sgs[b])

    def wait_gather(b):
        pltpu.make_async_copy(
            zs.at[idxv.at[pl.ds(0, C)]],
            abs_[b].at[pl.ds(0, C)], sgh[b]).wait()
        pltpu.make_async_copy(
            zs.at[idxv.at[pl.ds(0, C)]],
            abs_[b].at[pl.ds(C, C)], sgs[b]).wait()

    def wait_out(b):
        pltpu.make_async_copy(
            os_[b].at[pl.ds(0, C)], out_hbm.at[pl.ds(base, C)], sos[b]).wait()

    def compute(b):
        ab = abs_[b]
        o = os_[b]

        @plsc.parallel_loop(0, C, unroll=2)
        def edge(i):
            parts = [None] * 4
            for g in range(4):
                wa = ab[i, pl.ds(g * 16, 16)]
                wb = ab[i + C, pl.ds(g * 16, 16)]
                p = plsc.bitcast(wa, jnp.bfloat16) * plsc.bitcast(wb, jnp.bfloat16)
                p0, p1 = plsc.unpack(p, format=plsc.PackFormat.INTERLEAVED)
                parts[g] = p0 + p1
            acc = (parts[0] + parts[1]) + (parts[2] + parts[3])
            csum = plsc.cumsum(acc)
            plsc.store_scatter(o, [jnp.full((16,), 0, jnp.int32) + i], csum,
                               mask=last_lane)

        @plsc.parallel_loop(0, NBLK)
        def sig(k):
            v = o[pl.ds(k * 16, 16)]
            o[pl.ds(k * 16, 16)] = 1.0 / (1.0 + jnp.exp(-v))

    pltpu.async_copy(row_hbm.at[pl.ds(base, E_W)], idxv.at[pl.ds(0, E_W)],
                     semi)
    pltpu.async_copy(col_hbm.at[pl.ds(base, E_W)], idxv.at[pl.ds(E_W, E_W)],
                     semi)
    NR = N_NODES // 16
    pltpu.sync_copy(z_hbm.at[pl.ds(sid * NR, NR)], zs.at[pl.ds(sid * NR, NR)])
    pltpu.make_async_copy(row_hbm.at[pl.ds(0, 2 * E_W)], idxv, semi).wait()
    plsc.subcore_barrier()
    gathers(0, 0)

    def super_(si, _):
        for b in (0, 1):
            ci = si * 2 + b
            nb = 1 - b

            wait_gather(b)

            @pl.when(ci + 1 < NCHUNK)
            def _():
                gathers(ci + 1, nb)

            @pl.when(ci >= 2)
            def _():
                wait_out(b)

            compute(b)
            pltpu.async_copy(
                os_[b].at[pl.ds(0, C)],
                out_hbm.at[pl.ds(base + ci * C, C)], sos[b])
        return 0

    lax.fori_loop(0, NCHUNK // 2, super_, 0)
    wait_out(0)
    wait_out(1)


@jax.jit
def kernel(z, edge_index):
    row = edge_index[0].astype(jnp.int32)
    col = edge_index[1].astype(jnp.int32)
    zb = z.astype(jnp.bfloat16).reshape(N_NODES, DW, 2)
    zi = lax.bitcast_convert_type(zb, jnp.int32)
    mesh = plsc.VectorSubcoreMesh(core_axis_name="c", subcore_axis_name="s")
    f = functools.partial(
        pl.kernel,
        mesh=mesh,
        compiler_params=pltpu.CompilerParams(
            needs_layout_passes=False, use_tc_tiling_on_sc=False),
        out_type=jax.ShapeDtypeStruct((N_EDGES,), jnp.float32),
        scratch_types=[
            pltpu.VMEM((2 * E_W,), jnp.int32),
            pltpu.VMEM_SHARED((N_NODES, DW), jnp.int32),
            pltpu.VMEM((2 * C, DW), jnp.int32),
            pltpu.VMEM((2 * C, DW), jnp.int32),
            pltpu.VMEM((16 * NBLK,), jnp.float32),
            pltpu.VMEM((16 * NBLK,), jnp.float32),
            pltpu.SemaphoreType.DMA,
            pltpu.SemaphoreType.DMA,
            pltpu.SemaphoreType.DMA,
            pltpu.SemaphoreType.DMA,
            pltpu.SemaphoreType.DMA,
            pltpu.SemaphoreType.DMA,
            pltpu.SemaphoreType.DMA,
        ],
    )(_sc_kernel)
    return f(zi, row, col)
